# Initial kernel scaffold; baseline (speedup 1.0000x reference)
import functools

import jax
import jax.numpy as jnp
from jax import lax
from jax.experimental import pallas as pl
from jax.experimental.pallas import tpu as pltpu
from jax.experimental.pallas import tpu_sc as plsc

D = 128
NSEG = 1000
SEG_PAD = 1024
CHUNK = 100
NCHUNKS = 1000
NC = 2
NS = 16
NW = NC * NS
CNT_W = 16


def _sc_segment_sums(feat3d, idx2d):
    mesh = plsc.VectorSubcoreMesh(core_axis_name="c", subcore_axis_name="s")

    @functools.partial(
        pl.kernel,
        mesh=mesh,
        out_type=[
            jax.ShapeDtypeStruct((NC, SEG_PAD, D), jnp.float32),
            jax.ShapeDtypeStruct((NC, SEG_PAD, CNT_W), jnp.float32),
        ],
        scratch_types=[
            pltpu.VMEM((CHUNK, D), jnp.float32),
            pltpu.VMEM((CHUNK,), jnp.int32),
            pltpu.VMEM((CHUNK, CNT_W), jnp.float32),
            pltpu.VMEM((SEG_PAD // NS, CNT_W), jnp.float32),
            pltpu.VMEM_SHARED((SEG_PAD, D), jnp.float32),
            pltpu.VMEM_SHARED((SEG_PAD, CNT_W), jnp.float32),
        ],
    )
    def k(feat_hbm, idx_hbm, out_feat, out_cnt, buf, idx_v, ones_v, zc, acc,
          acc_cnt):
        c = lax.axis_index("c")
        s = lax.axis_index("s")
        wid = s * NC + c
        rows = SEG_PAD // NS

        def zrow(i, _):
            def zlane(j, _):
                buf[i, pl.ds(j * 16, 16)] = jnp.zeros((16,), jnp.float32)
                return 0
            return lax.fori_loop(0, D // 16, zlane, 0)

        lax.fori_loop(0, rows, zrow, 0)

        def zcrow(i, _):
            zc[i, :] = jnp.zeros((CNT_W,), jnp.float32)
            return 0

        lax.fori_loop(0, rows, zcrow, 0)

        def orow(i, _):
            ones_v[i, :] = jnp.ones((CNT_W,), jnp.float32)
            return 0

        lax.fori_loop(0, CHUNK, orow, 0)

        pltpu.sync_copy(buf.at[pl.ds(0, rows)], acc.at[pl.ds(s * rows, rows)])
        pltpu.sync_copy(zc, acc_cnt.at[pl.ds(s * rows, rows)])
        plsc.subcore_barrier()

        def body(kk, _):
            ch = wid + kk * NW

            @pl.when(ch < NCHUNKS)
            def _():
                pltpu.sync_copy(idx_hbm.at[ch], idx_v)
                pltpu.sync_copy(feat_hbm.at[ch], buf)
                pltpu.sync_copy(buf, acc.at[idx_v], add=True)
                pltpu.sync_copy(ones_v, acc_cnt.at[idx_v], add=True)

            return 0

        lax.fori_loop(0, (NCHUNKS + NW - 1) // NW, body, 0)
        plsc.subcore_barrier()

        @pl.when(s == 0)
        def _():
            pltpu.sync_copy(acc, out_feat.at[c])
            pltpu.sync_copy(acc_cnt, out_cnt.at[c])

    return k(feat3d, idx2d)


def _tc_finish(pf, pc, W, b2):
    def body(pf_ref, pc_ref, w_ref, b_ref, o_ref):
        seg = pf_ref[0] + pf_ref[1]
        cnt = pc_ref[0] + pc_ref[1]
        r = jnp.dot(seg, w_ref[...], preferred_element_type=jnp.float32)
        r = r + cnt[:, 0:1] * b_ref[...]
        o_ref[...] = r[:NSEG]

    return pl.pallas_call(
        body,
        out_shape=jax.ShapeDtypeStruct((NSEG, D), jnp.float32),
    )(pf, pc, W, b2)


def kernel(features, structural_indices, W, b):
    feat3d = features.reshape(NCHUNKS, CHUNK, D)
    idx2d = structural_indices.reshape(NCHUNKS, CHUNK)
    pf, pc = _sc_segment_sums(feat3d, idx2d)
    return _tc_finish(pf, pc, W, b.reshape(1, D))


# speedup vs baseline: 2.4930x; 2.4930x over previous
import functools

import jax
import jax.numpy as jnp
from jax import lax
from jax.experimental import pallas as pl
from jax.experimental.pallas import tpu as pltpu
from jax.experimental.pallas import tpu_sc as plsc

D = 128
NSEG = 1000
SEG_PAD = 1024
CHUNK = 100
NCHUNKS = 1000
NC = 2
NS = 16
NW = NC * NS


def _sc_segment_sums(feat3d, idx2d):
    mesh = plsc.VectorSubcoreMesh(core_axis_name="c", subcore_axis_name="s")

    @functools.partial(
        pl.kernel,
        mesh=mesh,
        out_type=[
            jax.ShapeDtypeStruct((NC, SEG_PAD, D), jnp.float32),
            jax.ShapeDtypeStruct((NC, SEG_PAD, D), jnp.float32),
        ],
        scratch_types=[
            pltpu.VMEM((CHUNK, D), jnp.float32),
            pltpu.VMEM((CHUNK, D), jnp.float32),
            pltpu.VMEM((CHUNK,), jnp.int32),
            pltpu.VMEM_SHARED((SEG_PAD, D), jnp.float32),
            pltpu.VMEM_SHARED((SEG_PAD, D), jnp.float32),
        ],
    )
    def k(feat_hbm, idx_hbm, out_feat, out_cnt, buf, ones_v, idx_v, acc,
          acc_cnt):
        c = lax.axis_index("c")
        s = lax.axis_index("s")
        wid = s * NC + c
        rows = SEG_PAD // NS

        def zrow(i, _):
            def zlane(j, _):
                buf[i, pl.ds(j * 16, 16)] = jnp.zeros((16,), jnp.float32)
                return 0
            return lax.fori_loop(0, D // 16, zlane, 0)

        lax.fori_loop(0, rows, zrow, 0)
        pltpu.sync_copy(buf.at[pl.ds(0, rows)], acc.at[pl.ds(s * rows, rows)])
        pltpu.sync_copy(buf.at[pl.ds(0, rows)],
                        acc_cnt.at[pl.ds(s * rows, rows)])

        def orow(i, _):
            def olane(j, _):
                ones_v[i, pl.ds(j * 16, 16)] = jnp.ones((16,), jnp.float32)
                return 0
            return lax.fori_loop(0, D // 16, olane, 0)

        lax.fori_loop(0, CHUNK, orow, 0)
        plsc.subcore_barrier()

        def body(kk, _):
            ch = wid + kk * NW

            @pl.when(ch < NCHUNKS)
            def _():
                pltpu.sync_copy(idx_hbm.at[ch], idx_v)
                pltpu.sync_copy(feat_hbm.at[ch], buf)
                pltpu.sync_copy(buf, acc.at[idx_v], add=True)
                pltpu.sync_copy(ones_v, acc_cnt.at[idx_v], add=True)

            return 0

        lax.fori_loop(0, (NCHUNKS + NW - 1) // NW, body, 0)
        plsc.subcore_barrier()

        @pl.when(s == 0)
        def _():
            pltpu.sync_copy(acc, out_feat.at[c])
            pltpu.sync_copy(acc_cnt, out_cnt.at[c])

    return k(feat3d, idx2d)


def _tc_finish(pf, pc, W, b2):
    def body(pf_ref, pc_ref, w_ref, b_ref, o_ref):
        seg = pf_ref[0] + pf_ref[1]
        cnt = pc_ref[0] + pc_ref[1]
        r = jnp.dot(seg, w_ref[...], preferred_element_type=jnp.float32)
        r = r + cnt[:, 0:1] * b_ref[...]
        o_ref[...] = r[:NSEG]

    return pl.pallas_call(
        body,
        out_shape=jax.ShapeDtypeStruct((NSEG, D), jnp.float32),
    )(pf, pc, W, b2)


def kernel(features, structural_indices, W, b):
    feat3d = features.reshape(NCHUNKS, CHUNK, D)
    idx2d = structural_indices.reshape(NCHUNKS, CHUNK)
    pf, pc = _sc_segment_sums(feat3d, idx2d)
    return _tc_finish(pf, pc, W, b.reshape(1, D))


# speedup vs baseline: 3.1805x; 1.2758x over previous
import functools

import jax
import jax.numpy as jnp
from jax import lax
from jax.experimental import pallas as pl
from jax.experimental.pallas import tpu as pltpu
from jax.experimental.pallas import tpu_sc as plsc

D = 128
NSEG = 1000
SEG_PAD = 1024
CHUNK = 100
NCHUNKS = 1000
NC = 2
NS = 16
NW = NC * NS
MAXCH = 32


def _sc_segment_sums(feat3d, idx2d):
    mesh = plsc.VectorSubcoreMesh(core_axis_name="c", subcore_axis_name="s")

    @functools.partial(
        pl.kernel,
        mesh=mesh,
        out_type=[
            jax.ShapeDtypeStruct((NC, SEG_PAD, D), jnp.float32),
            jax.ShapeDtypeStruct((NC, SEG_PAD, D), jnp.float32),
        ],
        scratch_types=[
            pltpu.VMEM((CHUNK, D), jnp.float32),
            pltpu.VMEM((CHUNK, D), jnp.float32),
            pltpu.VMEM((CHUNK, D), jnp.float32),
            pltpu.VMEM((CHUNK,), jnp.int32),
            pltpu.VMEM((CHUNK,), jnp.int32),
            pltpu.VMEM_SHARED((SEG_PAD, D), jnp.float32),
            pltpu.VMEM_SHARED((SEG_PAD, D), jnp.float32),
            pltpu.SemaphoreType.DMA,
            pltpu.SemaphoreType.DMA,
            pltpu.SemaphoreType.DMA,
            pltpu.SemaphoreType.DMA,
        ],
    )
    def k(feat_hbm, idx_hbm, out_feat, out_cnt, buf0, buf1, ones_v, idx0,
          idx1, acc, acc_cnt, sf0, sf1, si0, si1):
        c = lax.axis_index("c")
        s = lax.axis_index("s")
        wid = s * NC + c
        rows = SEG_PAD // NS
        start = 31 * wid + jnp.minimum(wid, 8)
        n = jnp.where(wid < 8, 32, 31)

        def zrow(i, _):
            def zlane(j, _):
                buf0[i, pl.ds(j * 16, 16)] = jnp.zeros((16,), jnp.float32)
                return 0
            return lax.fori_loop(0, D // 16, zlane, 0)

        lax.fori_loop(0, rows, zrow, 0)
        pltpu.sync_copy(buf0.at[pl.ds(0, rows)], acc.at[pl.ds(s * rows, rows)])
        pltpu.sync_copy(buf0.at[pl.ds(0, rows)],
                        acc_cnt.at[pl.ds(s * rows, rows)])

        def orow(i, _):
            def olane(j, _):
                ones_v[i, pl.ds(j * 16, 16)] = jnp.ones((16,), jnp.float32)
                return 0
            return lax.fori_loop(0, D // 16, olane, 0)

        lax.fori_loop(0, CHUNK, orow, 0)
        plsc.subcore_barrier()

        bufs = (buf0, buf1)
        idxs = (idx0, idx1)
        sfs = (sf0, sf1)
        sis = (si0, si1)

        def dma_pair(lc, slot):
            g = start + lc
            return (pltpu.make_async_copy(feat_hbm.at[g], bufs[slot],
                                          sfs[slot]),
                    pltpu.make_async_copy(idx_hbm.at[g], idxs[slot],
                                          sis[slot]))

        def fire(lc, slot):
            f, i = dma_pair(lc, slot)
            f.start()
            i.start()

        def drain(lc, slot):
            f, i = dma_pair(lc, slot)
            f.wait()
            i.wait()

        def scatter(slot):
            pltpu.sync_copy(bufs[slot], acc.at[idxs[slot]], add=True)
            pltpu.sync_copy(ones_v, acc_cnt.at[idxs[slot]], add=True)

        fire(0, 0)

        def body(p, _):
            c0 = 2 * p

            for q in range(2):
                cq = c0 + q

                @pl.when(cq < n)
                def _():
                    drain(cq, q)

                    @pl.when(cq + 1 < n)
                    def _():
                        fire(cq + 1, 1 - q)

                    scatter(q)

            return 0

        lax.fori_loop(0, MAXCH // 2, body, 0)
        plsc.subcore_barrier()

        @pl.when(s == 0)
        def _():
            pltpu.sync_copy(acc, out_feat.at[c])
            pltpu.sync_copy(acc_cnt, out_cnt.at[c])

    return k(feat3d, idx2d)


def _tc_finish(pf, pc, W, b2):
    def body(pf_ref, pc_ref, w_ref, b_ref, o_ref):
        seg = pf_ref[0] + pf_ref[1]
        cnt = pc_ref[0] + pc_ref[1]
        r = jnp.dot(seg, w_ref[...], preferred_element_type=jnp.float32)
        r = r + cnt[:, 0:1] * b_ref[...]
        o_ref[...] = r[:NSEG]

    return pl.pallas_call(
        body,
        out_shape=jax.ShapeDtypeStruct((NSEG, D), jnp.float32),
    )(pf, pc, W, b2)


def kernel(features, structural_indices, W, b):
    feat3d = features.reshape(NCHUNKS, CHUNK, D)
    idx2d = structural_indices.reshape(NCHUNKS, CHUNK)
    pf, pc = _sc_segment_sums(feat3d, idx2d)
    return _tc_finish(pf, pc, W, b.reshape(1, D))


# speedup vs baseline: 5.4196x; 1.7040x over previous
import functools

import jax
import jax.numpy as jnp
from jax import lax
from jax.experimental import pallas as pl
from jax.experimental.pallas import tpu as pltpu
from jax.experimental.pallas import tpu_sc as plsc

D = 128
NSEG = 1000
SEG_PAD = 1024
CHUNK = 80
NCHUNKS = 1250
NC = 2
NS = 16
NW = NC * NS
MAXCH = 40


def _sc_segment_sums(features, idx):
    mesh = plsc.VectorSubcoreMesh(core_axis_name="c", subcore_axis_name="s")

    @functools.partial(
        pl.kernel,
        mesh=mesh,
        out_type=[
            jax.ShapeDtypeStruct((NC, SEG_PAD, D), jnp.float32),
            jax.ShapeDtypeStruct((NC, SEG_PAD, D), jnp.float32),
        ],
        scratch_types=[
            pltpu.VMEM((CHUNK, D), jnp.float32),
            pltpu.VMEM((CHUNK, D), jnp.float32),
            pltpu.VMEM((CHUNK, D), jnp.float32),
            pltpu.VMEM((CHUNK,), jnp.int32),
            pltpu.VMEM((CHUNK,), jnp.int32),
            pltpu.VMEM_SHARED((SEG_PAD, D), jnp.float32),
            pltpu.VMEM_SHARED((SEG_PAD, D), jnp.float32),
            pltpu.SemaphoreType.DMA,
            pltpu.SemaphoreType.DMA,
            pltpu.SemaphoreType.DMA,
            pltpu.SemaphoreType.DMA,
        ],
    )
    def k(feat_hbm, idx_hbm, out_feat, out_cnt, buf0, buf1, ones_v, idx0,
          idx1, acc, acc_cnt, sf0, sf1, si0, si1):
        c = lax.axis_index("c")
        s = lax.axis_index("s")
        wid = s * NC + c
        rows = SEG_PAD // NS
        start = 39 * wid + jnp.minimum(wid, 2)
        n = jnp.where(wid < 2, 40, 39)

        def zrow(i, _):
            def zlane(j, _):
                buf0[i, pl.ds(j * 16, 16)] = jnp.zeros((16,), jnp.float32)
                return 0
            return lax.fori_loop(0, D // 16, zlane, 0)

        lax.fori_loop(0, rows, zrow, 0)
        pltpu.sync_copy(buf0.at[pl.ds(0, rows)], acc.at[pl.ds(s * rows, rows)])
        pltpu.sync_copy(buf0.at[pl.ds(0, rows)],
                        acc_cnt.at[pl.ds(s * rows, rows)])

        def orow(i, _):
            def olane(j, _):
                ones_v[i, pl.ds(j * 16, 16)] = jnp.ones((16,), jnp.float32)
                return 0
            return lax.fori_loop(0, D // 16, olane, 0)

        lax.fori_loop(0, CHUNK, orow, 0)
        plsc.subcore_barrier()

        bufs = (buf0, buf1)
        idxs = (idx0, idx1)
        sfs = (sf0, sf1)
        sis = (si0, si1)

        def dma_pair(lc, slot):
            a = (start + lc) * CHUNK
            return (pltpu.make_async_copy(feat_hbm.at[pl.ds(a, CHUNK)],
                                          bufs[slot], sfs[slot]),
                    pltpu.make_async_copy(idx_hbm.at[pl.ds(a, CHUNK)],
                                          idxs[slot], sis[slot]))

        def fire(lc, slot):
            f, i = dma_pair(lc, slot)
            f.start()
            i.start()

        def drain(lc, slot):
            f, i = dma_pair(lc, slot)
            f.wait()
            i.wait()

        def scatter(slot):
            pltpu.sync_copy(bufs[slot], acc.at[idxs[slot]], add=True)
            pltpu.sync_copy(ones_v, acc_cnt.at[idxs[slot]], add=True)

        fire(0, 0)

        def body(p, _):
            c0 = 2 * p

            for q in range(2):
                cq = c0 + q

                @pl.when(cq < n)
                def _():
                    drain(cq, q)

                    @pl.when(cq + 1 < n)
                    def _():
                        fire(cq + 1, 1 - q)

                    scatter(q)

            return 0

        lax.fori_loop(0, MAXCH // 2, body, 0)
        plsc.subcore_barrier()

        @pl.when(s == 0)
        def _():
            pltpu.sync_copy(acc, out_feat.at[c])
            pltpu.sync_copy(acc_cnt, out_cnt.at[c])

    return k(features, idx)


def _tc_finish(pf, pc, W, b2):
    def body(pf_ref, pc_ref, w_ref, b_ref, o_ref):
        seg = pf_ref[0] + pf_ref[1]
        cnt = pc_ref[0] + pc_ref[1]
        r = jnp.dot(seg, w_ref[...], preferred_element_type=jnp.float32)
        r = r + cnt[:, 0:1] * b_ref[...]
        o_ref[...] = r[:NSEG]

    return pl.pallas_call(
        body,
        out_shape=jax.ShapeDtypeStruct((NSEG, D), jnp.float32),
    )(pf, pc, W, b2)


def kernel(features, structural_indices, W, b):
    pf, pc = _sc_segment_sums(features, structural_indices)
    return _tc_finish(pf, pc, W, b.reshape(1, D))
